# Initial kernel scaffold; baseline (speedup 1.0000x reference)
import jax
import jax.numpy as jnp
from jax.experimental import pallas as pl


def kernel(x, W, b):
    raise NotImplementedError("write your pallas kernel here")


# speedup vs baseline: 3.1736x; 3.1736x over previous
import functools

import jax
import jax.numpy as jnp
from jax.experimental import pallas as pl

D_MODEL = 768
NUM_EXPERTS = 8
TOP_K = 2
BLOCK = 2048


def _router_block(x_ref, w_ref, b_ref, mix_ref, probs_ref, idx_ref, tw_ref):
    x = x_ref[...]
    logits = jnp.dot(x, w_ref[...].T, preferred_element_type=jnp.float32)
    logits = logits + b_ref[...]

    m = jnp.max(logits, axis=1, keepdims=True)
    e = jnp.exp(logits - m)
    s = jnp.sum(e, axis=1, keepdims=True)
    probs = e / s

    ecols = jax.lax.broadcasted_iota(jnp.int32, probs.shape, 1)
    v1 = jnp.max(probs, axis=1, keepdims=True)
    i1 = jnp.min(jnp.where(probs == v1, ecols, NUM_EXPERTS), axis=1, keepdims=True)
    masked = jnp.where(ecols == i1, -1.0, probs)
    v2 = jnp.max(masked, axis=1, keepdims=True)
    i2 = jnp.min(jnp.where(masked == v2, ecols, NUM_EXPERTS), axis=1, keepdims=True)

    denom = v1 + v2 + 1e-9
    w1 = v1 / denom
    w2 = v2 / denom

    zero = jnp.zeros_like(probs)
    mixing = jnp.where(ecols == i1, w1, zero) + jnp.where(ecols == i2, w2, zero)

    mix_ref[...] = mixing
    probs_ref[...] = probs
    idx_ref[...] = jnp.concatenate([i1, i2], axis=1)
    tw_ref[...] = jnp.concatenate([w1, w2], axis=1)


@functools.partial(jax.jit, static_argnames=())
def kernel(x, W, b):
    n, d = x.shape
    e = W.shape[0]
    b2 = b.reshape(1, e)
    grid = (n // BLOCK,)
    out = pl.pallas_call(
        _router_block,
        grid=grid,
        in_specs=[
            pl.BlockSpec((BLOCK, d), lambda i: (i, 0)),
            pl.BlockSpec((e, d), lambda i: (0, 0)),
            pl.BlockSpec((1, e), lambda i: (0, 0)),
        ],
        out_specs=[
            pl.BlockSpec((BLOCK, e), lambda i: (i, 0)),
            pl.BlockSpec((BLOCK, e), lambda i: (i, 0)),
            pl.BlockSpec((BLOCK, TOP_K), lambda i: (i, 0)),
            pl.BlockSpec((BLOCK, TOP_K), lambda i: (i, 0)),
        ],
        out_shape=[
            jax.ShapeDtypeStruct((n, e), jnp.float32),
            jax.ShapeDtypeStruct((n, e), jnp.float32),
            jax.ShapeDtypeStruct((n, TOP_K), jnp.int32),
            jax.ShapeDtypeStruct((n, TOP_K), jnp.float32),
        ],
    )(x, W, b2)
    mixing, probs, idx, tw = out
    return (mixing, probs, idx, tw)


# speedup vs baseline: 3.1990x; 1.0080x over previous
import functools

import jax
import jax.numpy as jnp
from jax.experimental import pallas as pl
from jax.experimental.pallas import tpu as pltpu

D_MODEL = 768
NUM_EXPERTS = 8
TOP_K = 2
BLOCK = 2048


def _router_block(x_ref, w_ref, b_ref, mix_ref, probs_ref, idx_ref, tw_ref):
    x = x_ref[...]
    logits = jnp.dot(x, w_ref[...].T, preferred_element_type=jnp.float32)
    logits = logits + b_ref[...]

    m = jnp.max(logits, axis=1, keepdims=True)
    e = jnp.exp(logits - m)
    s = jnp.sum(e, axis=1, keepdims=True)
    probs = e / s

    ecols = jax.lax.broadcasted_iota(jnp.int32, probs.shape, 1)
    v1 = jnp.max(probs, axis=1, keepdims=True)
    i1 = jnp.min(jnp.where(probs == v1, ecols, NUM_EXPERTS), axis=1, keepdims=True)
    masked = jnp.where(ecols == i1, -1.0, probs)
    v2 = jnp.max(masked, axis=1, keepdims=True)
    i2 = jnp.min(jnp.where(masked == v2, ecols, NUM_EXPERTS), axis=1, keepdims=True)

    denom = v1 + v2 + 1e-9
    w1 = v1 / denom
    w2 = v2 / denom

    zero = jnp.zeros_like(probs)
    mixing = jnp.where(ecols == i1, w1, zero) + jnp.where(ecols == i2, w2, zero)

    mix_ref[...] = mixing
    probs_ref[...] = probs
    idx_ref[...] = jnp.concatenate([i1, i2], axis=1)
    tw_ref[...] = jnp.concatenate([w1, w2], axis=1)


@functools.partial(jax.jit, static_argnames=())
def kernel(x, W, b):
    n, d = x.shape
    e = W.shape[0]
    b2 = b.reshape(1, e)
    grid = (n // BLOCK,)
    out = pl.pallas_call(
        _router_block,
        grid=grid,
        in_specs=[
            pl.BlockSpec((BLOCK, d), lambda i: (i, 0)),
            pl.BlockSpec((e, d), lambda i: (0, 0)),
            pl.BlockSpec((1, e), lambda i: (0, 0)),
        ],
        out_specs=[
            pl.BlockSpec((BLOCK, e), lambda i: (i, 0)),
            pl.BlockSpec((BLOCK, e), lambda i: (i, 0)),
            pl.BlockSpec((BLOCK, TOP_K), lambda i: (i, 0)),
            pl.BlockSpec((BLOCK, TOP_K), lambda i: (i, 0)),
        ],
        out_shape=[
            jax.ShapeDtypeStruct((n, e), jnp.float32),
            jax.ShapeDtypeStruct((n, e), jnp.float32),
            jax.ShapeDtypeStruct((n, TOP_K), jnp.int32),
            jax.ShapeDtypeStruct((n, TOP_K), jnp.float32),
        ],
        compiler_params=pltpu.CompilerParams(
            dimension_semantics=("parallel",),
        ),
    )(x, W, b2)
    mixing, probs, idx, tw = out
    return (mixing, probs, idx, tw)


# speedup vs baseline: 3.2906x; 1.0286x over previous
import functools

import jax
import jax.numpy as jnp
from jax.experimental import pallas as pl
from jax.experimental.pallas import tpu as pltpu

D_MODEL = 768
NUM_EXPERTS = 8
TOP_K = 2
BLOCK = 4096


def _router_block(x_ref, w_ref, b_ref, mix_ref, probs_ref, idx_ref, tw_ref):
    x = x_ref[...]
    logits = jnp.dot(x, w_ref[...].T, preferred_element_type=jnp.float32)
    logits = logits + b_ref[...]

    m = jnp.max(logits, axis=1, keepdims=True)
    e = jnp.exp(logits - m)
    s = jnp.sum(e, axis=1, keepdims=True)
    probs = e / s

    ecols = jax.lax.broadcasted_iota(jnp.int32, probs.shape, 1)
    v1 = jnp.max(probs, axis=1, keepdims=True)
    i1 = jnp.min(jnp.where(probs == v1, ecols, NUM_EXPERTS), axis=1, keepdims=True)
    masked = jnp.where(ecols == i1, -1.0, probs)
    v2 = jnp.max(masked, axis=1, keepdims=True)
    i2 = jnp.min(jnp.where(masked == v2, ecols, NUM_EXPERTS), axis=1, keepdims=True)

    denom = v1 + v2 + 1e-9
    w1 = v1 / denom
    w2 = v2 / denom

    zero = jnp.zeros_like(probs)
    mixing = jnp.where(ecols == i1, w1, zero) + jnp.where(ecols == i2, w2, zero)

    mix_ref[...] = mixing
    probs_ref[...] = probs
    idx_ref[...] = jnp.concatenate([i1, i2], axis=1)
    tw_ref[...] = jnp.concatenate([w1, w2], axis=1)


@functools.partial(jax.jit, static_argnames=())
def kernel(x, W, b):
    n, d = x.shape
    e = W.shape[0]
    b2 = b.reshape(1, e)
    grid = (n // BLOCK,)
    out = pl.pallas_call(
        _router_block,
        grid=grid,
        in_specs=[
            pl.BlockSpec((BLOCK, d), lambda i: (i, 0)),
            pl.BlockSpec((e, d), lambda i: (0, 0)),
            pl.BlockSpec((1, e), lambda i: (0, 0)),
        ],
        out_specs=[
            pl.BlockSpec((BLOCK, e), lambda i: (i, 0)),
            pl.BlockSpec((BLOCK, e), lambda i: (i, 0)),
            pl.BlockSpec((BLOCK, TOP_K), lambda i: (i, 0)),
            pl.BlockSpec((BLOCK, TOP_K), lambda i: (i, 0)),
        ],
        out_shape=[
            jax.ShapeDtypeStruct((n, e), jnp.float32),
            jax.ShapeDtypeStruct((n, e), jnp.float32),
            jax.ShapeDtypeStruct((n, TOP_K), jnp.int32),
            jax.ShapeDtypeStruct((n, TOP_K), jnp.float32),
        ],
        compiler_params=pltpu.CompilerParams(
            dimension_semantics=("parallel",),
        ),
    )(x, W, b2)
    mixing, probs, idx, tw = out
    return (mixing, probs, idx, tw)


# speedup vs baseline: 3.4540x; 1.0497x over previous
import functools

import jax
import jax.numpy as jnp
from jax.experimental import pallas as pl
from jax.experimental.pallas import tpu as pltpu

D_MODEL = 768
NUM_EXPERTS = 8
TOP_K = 2
BLOCK = 4096


def _router_block(x_ref, w_ref, b_ref, mix_ref, probs_ref, idx_ref, tw_ref):
    x = x_ref[...]
    w = w_ref[...]
    logits = jax.lax.dot_general(
        w, x, (((1,), (1,)), ((), ())),
        preferred_element_type=jnp.float32,
    ) + b_ref[...]

    m = jnp.max(logits, axis=0, keepdims=True)
    e = jnp.exp(logits - m)
    s = jnp.sum(e, axis=0, keepdims=True)
    probs = e * (1.0 / s)

    erows = jax.lax.broadcasted_iota(jnp.int32, e.shape, 0)
    v1 = jnp.max(e, axis=0, keepdims=True)
    i1 = jnp.min(jnp.where(e == v1, erows, NUM_EXPERTS), axis=0, keepdims=True)
    masked = jnp.where(erows == i1, -1.0, e)
    v2 = jnp.max(masked, axis=0, keepdims=True)
    i2 = jnp.min(jnp.where(masked == v2, erows, NUM_EXPERTS), axis=0, keepdims=True)

    inv = 1.0 / (v1 + v2)
    w1 = v1 * inv
    w2 = v2 * inv

    zero = jnp.zeros_like(e)
    mixing = jnp.where(erows == i1, w1, zero) + jnp.where(erows == i2, w2, zero)

    mix_ref[...] = mixing.T
    probs_ref[...] = probs.T
    idx_ref[...] = jnp.concatenate([i1, i2], axis=0).T
    tw_ref[...] = jnp.concatenate([w1, w2], axis=0).T


@functools.partial(jax.jit, static_argnames=())
def kernel(x, W, b):
    n, d = x.shape
    e = W.shape[0]
    b2 = b.reshape(e, 1)
    grid = (n // BLOCK,)
    out = pl.pallas_call(
        _router_block,
        grid=grid,
        in_specs=[
            pl.BlockSpec((BLOCK, d), lambda i: (i, 0)),
            pl.BlockSpec((e, d), lambda i: (0, 0)),
            pl.BlockSpec((e, 1), lambda i: (0, 0)),
        ],
        out_specs=[
            pl.BlockSpec((BLOCK, e), lambda i: (i, 0)),
            pl.BlockSpec((BLOCK, e), lambda i: (i, 0)),
            pl.BlockSpec((BLOCK, TOP_K), lambda i: (i, 0)),
            pl.BlockSpec((BLOCK, TOP_K), lambda i: (i, 0)),
        ],
        out_shape=[
            jax.ShapeDtypeStruct((n, e), jnp.float32),
            jax.ShapeDtypeStruct((n, e), jnp.float32),
            jax.ShapeDtypeStruct((n, TOP_K), jnp.int32),
            jax.ShapeDtypeStruct((n, TOP_K), jnp.float32),
        ],
        compiler_params=pltpu.CompilerParams(
            dimension_semantics=("parallel",),
        ),
    )(x, W, b2)
    mixing, probs, idx, tw = out
    return (mixing, probs, idx, tw)
